# baseline (speedup 1.0000x reference)
import functools

import jax
import jax.numpy as jnp
from jax import lax
from jax.experimental import pallas as pl
from jax.experimental.pallas import tpu as pltpu
from jax.experimental.pallas import tpu_sc as plsc

N_NODES = 10000
N_EDGES = 320000
D_EDGE = 16
N_PAD = 10240

NC = 2
NS = 16
NW = NC * NS

B = 80
G_TOTAL = N_EDGES // B
CHUNK_G = 32
CHUNK_E = CHUNK_G * B
N_CHUNKS = G_TOTAL // CHUNK_G
MAX_ROUNDS = -(-N_CHUNKS // NW)
RPS = N_PAD // NS

_mesh = plsc.VectorSubcoreMesh(core_axis_name="c", subcore_axis_name="s")


@functools.partial(
    pl.kernel,
    out_type=jax.ShapeDtypeStruct((NC, N_PAD, D_EDGE), jnp.bfloat16),
    mesh=_mesh,
    compiler_params=pltpu.CompilerParams(use_tc_tiling_on_sc=False),
    scratch_types=[
        pltpu.VMEM((CHUNK_G, B), jnp.int32),
        pltpu.VMEM((CHUNK_G, B), jnp.int32),
        pltpu.VMEM((CHUNK_G, B, D_EDGE), jnp.bfloat16),
        pltpu.VMEM((CHUNK_G, B, D_EDGE), jnp.bfloat16),
        pltpu.VMEM((RPS, D_EDGE), jnp.bfloat16),
        pltpu.VMEM_SHARED((N_PAD, D_EDGE), jnp.bfloat16),
        pltpu.SemaphoreType.DMA,
        pltpu.SemaphoreType.DMA,
        pltpu.SemaphoreType.DMA,
    ],
)
def _scatter_partials(col2d, attr, partials, idx_v0, idx_v1, attr_v0, attr_v1,
                      zbuf, acc, sem_l0, sem_l1, sem_s):
    c = lax.axis_index("c")
    s = lax.axis_index("s")
    w = s * NC + c
    sem_l = (sem_l0, sem_l1)
    idx_v = (idx_v0, idx_v1)
    attr_v = (attr_v0, attr_v1)

    zero2x16 = jnp.zeros((2, D_EDGE), jnp.bfloat16)

    def _zero(i, carry):
        zbuf[pl.ds(i * 2, 2), :] = zero2x16
        return carry

    lax.fori_loop(0, RPS // 2, _zero, 0)
    pltpu.sync_copy(zbuf, acc.at[pl.ds(s * RPS, RPS), :])
    plsc.subcore_barrier()

    def _fire_loads(rnd, buf):
        t = rnd * NW + w

        @pl.when(t < N_CHUNKS)
        def _():
            pltpu.async_copy(col2d.at[pl.ds(t * CHUNK_G, CHUNK_G), :],
                             idx_v[buf], sem_l[buf])
            pltpu.async_copy(attr.at[pl.ds(t * CHUNK_G, CHUNK_G), :, :],
                             attr_v[buf], sem_l[buf])

    _fire_loads(0, 0)
    for rnd in range(MAX_ROUNDS):
        b = rnd % 2
        t = rnd * NW + w
        if rnd + 1 < MAX_ROUNDS:
            _fire_loads(rnd + 1, 1 - b)

        @pl.when(t < N_CHUNKS)
        def _():
            pltpu.make_async_copy(col2d.at[pl.ds(0, CHUNK_G), :],
                                  idx_v[b], sem_l[b]).wait()
            pltpu.make_async_copy(attr.at[pl.ds(0, CHUNK_G), :, :],
                                  attr_v[b], sem_l[b]).wait()

            def _scat(j, carry):
                pltpu.async_copy(attr_v[b].at[j],
                                 acc.at[idx_v[b].at[j]], sem_s, add=True)
                return carry

            lax.fori_loop(0, CHUNK_G, _scat, 0)
            pltpu.make_async_copy(attr.at[pl.ds(0, CHUNK_G), :, :],
                                  attr_v[b], sem_s).wait()

    plsc.subcore_barrier()
    pltpu.sync_copy(acc.at[pl.ds(s * RPS, RPS), :],
                    partials.at[c, pl.ds(s * RPS, RPS), :])


CR = N_PAD // NW


@functools.partial(
    pl.kernel,
    out_type=jax.ShapeDtypeStruct((N_PAD, D_EDGE), jnp.bfloat16),
    mesh=_mesh,
    compiler_params=pltpu.CompilerParams(use_tc_tiling_on_sc=False),
    scratch_types=[
        pltpu.VMEM((CR, D_EDGE), jnp.bfloat16),
        pltpu.VMEM((CR, D_EDGE), jnp.bfloat16),
        pltpu.SemaphoreType.DMA,
    ],
)
def _combine(partials, out, a_v, b_v, sem):
    c = lax.axis_index("c")
    s = lax.axis_index("s")
    w = s * NC + c
    r0 = w * CR

    pltpu.async_copy(partials.at[0, pl.ds(r0, CR), :], a_v, sem)
    pltpu.async_copy(partials.at[1, pl.ds(r0, CR), :], b_v, sem)
    pltpu.make_async_copy(partials.at[0, pl.ds(r0, CR), :], a_v, sem).wait()
    pltpu.make_async_copy(partials.at[1, pl.ds(r0, CR), :], b_v, sem).wait()

    def _add(i, carry):
        r = pl.ds(i * 2, 2)
        a_v[r, :] = a_v[r, :] + b_v[r, :]
        return carry

    lax.fori_loop(0, CR // 2, _add, 0)
    pltpu.sync_copy(a_v, out.at[pl.ds(r0, CR), :])


def kernel(x, edge_index, edge_attr, u, batch):
    col = edge_index[1].astype(jnp.int32).reshape(G_TOTAL, B)
    attr3 = edge_attr.astype(jnp.bfloat16).reshape(G_TOTAL, B, D_EDGE)
    partials = _scatter_partials(col, attr3)
    return _combine(partials).astype(jnp.float32)[:N_NODES]


# speedup vs baseline: 1.6605x; 1.6605x over previous
import functools

import jax
import jax.numpy as jnp
from jax import lax
from jax.experimental import pallas as pl
from jax.experimental.pallas import tpu as pltpu
from jax.experimental.pallas import tpu_sc as plsc

N_NODES = 10000
N_EDGES = 320000
D_EDGE = 16
N_PAD = 10240

NC = 2
NS = 16
NW = NC * NS

ET = 128
T_TOTAL = N_EDGES // ET
TQ = 10
CHUNK_E = TQ * ET
N_CHUNKS = T_TOTAL // TQ
MAX_ROUNDS = -(-N_CHUNKS // NW)
RPS = N_PAD // NS

_mesh = plsc.VectorSubcoreMesh(core_axis_name="c", subcore_axis_name="s")


@functools.partial(
    pl.kernel,
    out_type=jax.ShapeDtypeStruct((NC, N_PAD, D_EDGE), jnp.float32),
    mesh=_mesh,
    compiler_params=pltpu.CompilerParams(use_tc_tiling_on_sc=False,
                                         needs_layout_passes=False),
    scratch_types=[
        pltpu.VMEM((TQ, ET), jnp.int32),
        pltpu.VMEM((TQ, ET), jnp.int32),
        pltpu.VMEM((TQ, 8, ET), jnp.float32),
        pltpu.VMEM((TQ, 8, ET), jnp.float32),
        pltpu.VMEM((TQ, 8, ET), jnp.float32),
        pltpu.VMEM((TQ, 8, ET), jnp.float32),
        pltpu.VMEM((CHUNK_E, D_EDGE), jnp.float32),
        pltpu.VMEM((CHUNK_E, D_EDGE), jnp.float32),
        pltpu.VMEM((RPS, D_EDGE), jnp.float32),
        pltpu.VMEM_SHARED((N_PAD, D_EDGE), jnp.float32),
        pltpu.SemaphoreType.DMA,
        pltpu.SemaphoreType.DMA,
        pltpu.SemaphoreType.DMA,
    ],
)
def _scatter_partials(col2d, att, partials, idx_v0, idx_v1, sa0, sa1, sb0, sb1,
                      tb0, tb1, zbuf, acc, sem_l0, sem_l1, sem_s):
    c = lax.axis_index("c")
    s = lax.axis_index("s")
    w = s * NC + c
    sem_l = (sem_l0, sem_l1)
    idx_v = (idx_v0, idx_v1)
    slab = ((sa0, sb0), (sa1, sb1))
    tbuf = (tb0, tb1)

    zero16 = jnp.zeros((D_EDGE,), jnp.float32)

    def _zero(i, carry):
        zbuf[i, :] = zero16
        return carry

    lax.fori_loop(0, RPS, _zero, 0)
    pltpu.sync_copy(zbuf, acc.at[pl.ds(s * RPS, RPS), :])
    plsc.subcore_barrier()

    iota16 = lax.iota(jnp.int32, 16)

    def _fire_loads(rnd, buf):
        t = rnd * NW + w

        @pl.when(t < N_CHUNKS)
        def _():
            t0 = t * TQ
            pltpu.async_copy(col2d.at[pl.ds(t0, TQ), :], idx_v[buf], sem_l[buf])
            pltpu.async_copy(att.at[pl.ds(t0, TQ), :, :],
                             slab[buf][0], sem_l[buf])
            pltpu.async_copy(att.at[pl.ds(T_TOTAL + t0, TQ), :, :],
                             slab[buf][1], sem_l[buf])

    def _wait_loads(buf):
        pltpu.make_async_copy(col2d.at[pl.ds(0, TQ), :],
                              idx_v[buf], sem_l[buf]).wait()
        pltpu.make_async_copy(att.at[pl.ds(0, TQ), :, :],
                              slab[buf][0], sem_l[buf]).wait()
        pltpu.make_async_copy(att.at[pl.ds(0, TQ), :, :],
                              slab[buf][1], sem_l[buf]).wait()

    def _detile(buf):
        for a in range(2):
            sl = slab[buf][a]
            tb = tbuf[buf]

            def _blk(u, carry):
                t = u // 8
                b = u - t * 8
                fcol = jnp.full((16,), 8 * a + b, jnp.int32)
                for k in range(8):
                    v = sl[t, b, pl.ds(k * 16, 16)]
                    rows = iota16 + (t * ET + k * 16)
                    plsc.store_scatter(tb, [rows, fcol], v)
                return carry

            lax.fori_loop(0, TQ * 8, _blk, 0)

    def _fire_scatters(buf):
        def _scat(q, carry):
            pltpu.async_copy(tbuf[buf].at[pl.ds(q * ET, ET), :],
                             acc.at[idx_v[buf].at[q]], sem_s, add=True)
            return carry

        lax.fori_loop(0, TQ, _scat, 0)

    def _drain_scatters(buf):
        pltpu.make_async_copy(att.at[pl.ds(0, TQ), :, :],
                              tbuf[buf], sem_s).wait()

    _fire_loads(0, 0)
    for rnd in range(MAX_ROUNDS):
        b = rnd % 2
        t = rnd * NW + w

        @pl.when(t < N_CHUNKS)
        def _():
            _wait_loads(b)
            _detile(b)

        if rnd > 0:
            tp = (rnd - 1) * NW + w

            @pl.when(tp < N_CHUNKS)
            def _():
                _drain_scatters(1 - b)

        if rnd + 1 < MAX_ROUNDS:
            _fire_loads(rnd + 1, 1 - b)

        @pl.when(t < N_CHUNKS)
        def _():
            _fire_scatters(b)

    tl = (MAX_ROUNDS - 1) * NW + w

    @pl.when(tl < N_CHUNKS)
    def _():
        _drain_scatters((MAX_ROUNDS - 1) % 2)

    plsc.subcore_barrier()
    pltpu.sync_copy(acc.at[pl.ds(s * RPS, RPS), :],
                    partials.at[c, pl.ds(s * RPS, RPS), :])


CR = N_PAD // NW


@functools.partial(
    pl.kernel,
    out_type=jax.ShapeDtypeStruct((N_PAD, D_EDGE), jnp.float32),
    mesh=_mesh,
    compiler_params=pltpu.CompilerParams(use_tc_tiling_on_sc=False),
    scratch_types=[
        pltpu.VMEM((CR, D_EDGE), jnp.float32),
        pltpu.VMEM((CR, D_EDGE), jnp.float32),
        pltpu.SemaphoreType.DMA,
    ],
)
def _combine(partials, out, a_v, b_v, sem):
    c = lax.axis_index("c")
    s = lax.axis_index("s")
    w = s * NC + c
    r0 = w * CR

    pltpu.async_copy(partials.at[0, pl.ds(r0, CR), :], a_v, sem)
    pltpu.async_copy(partials.at[1, pl.ds(r0, CR), :], b_v, sem)
    pltpu.make_async_copy(partials.at[0, pl.ds(r0, CR), :], a_v, sem).wait()
    pltpu.make_async_copy(partials.at[1, pl.ds(r0, CR), :], b_v, sem).wait()

    def _add(i, carry):
        a_v[i, :] = a_v[i, :] + b_v[i, :]
        return carry

    lax.fori_loop(0, CR, _add, 0)
    pltpu.sync_copy(a_v, out.at[pl.ds(r0, CR), :])


def kernel(x, edge_index, edge_attr, u, batch):
    col = edge_index[1].astype(jnp.int32).reshape(T_TOTAL, ET)
    att = (edge_attr.T.reshape(2, 8, T_TOTAL, ET)
           .transpose(0, 2, 1, 3).reshape(2 * T_TOTAL, 8, ET))
    partials = _scatter_partials(col, att)
    return _combine(partials)[:N_NODES]


# speedup vs baseline: 1.8623x; 1.1216x over previous
import functools

import jax
import jax.numpy as jnp
from jax import lax
from jax.experimental import pallas as pl
from jax.experimental.pallas import tpu as pltpu
from jax.experimental.pallas import tpu_sc as plsc

N_NODES = 10000
N_EDGES = 320000
D_EDGE = 16
N_PAD = 10240

NC = 2
NS = 16
NW = NC * NS

ET = 128
T_TOTAL = N_EDGES // ET
TQ = 10
CHUNK_E = TQ * ET
N_CHUNKS = T_TOTAL // TQ
MAX_ROUNDS = -(-N_CHUNKS // NW)
RPS = N_PAD // NS

_mesh = plsc.VectorSubcoreMesh(core_axis_name="c", subcore_axis_name="s")


@functools.partial(
    pl.kernel,
    out_type=jax.ShapeDtypeStruct((NC, N_PAD, D_EDGE), jnp.float32),
    mesh=_mesh,
    compiler_params=pltpu.CompilerParams(use_tc_tiling_on_sc=False,
                                         needs_layout_passes=False),
    scratch_types=[
        pltpu.VMEM((TQ, ET), jnp.int32),
        pltpu.VMEM((TQ, ET), jnp.int32),
        pltpu.VMEM((TQ, 8, ET), jnp.float32),
        pltpu.VMEM((TQ, 8, ET), jnp.float32),
        pltpu.VMEM((TQ, 8, ET), jnp.float32),
        pltpu.VMEM((TQ, 8, ET), jnp.float32),
        pltpu.VMEM((CHUNK_E, D_EDGE), jnp.float32),
        pltpu.VMEM((CHUNK_E, D_EDGE), jnp.float32),
        pltpu.VMEM((RPS, D_EDGE), jnp.float32),
        pltpu.VMEM_SHARED((N_PAD, D_EDGE), jnp.float32),
        pltpu.SemaphoreType.DMA,
        pltpu.SemaphoreType.DMA,
        pltpu.SemaphoreType.DMA,
    ],
)
def _scatter_partials(eidx, att, partials, idx_v0, idx_v1, sa0, sa1, sb0, sb1,
                      tb0, tb1, zbuf, acc, sem_l0, sem_l1, sem_s):
    c = lax.axis_index("c")
    s = lax.axis_index("s")
    w = s * NC + c
    sem_l = (sem_l0, sem_l1)
    idx_v = (idx_v0, idx_v1)
    slab = ((sa0, sb0), (sa1, sb1))
    tbuf = (tb0, tb1)

    zero16 = jnp.zeros((D_EDGE,), jnp.float32)

    def _zero(i, carry):
        zbuf[i, :] = zero16
        return carry

    lax.fori_loop(0, RPS, _zero, 0)
    pltpu.sync_copy(zbuf, acc.at[pl.ds(s * RPS, RPS), :])
    plsc.subcore_barrier()

    iota16 = lax.iota(jnp.int32, 16)

    def _fire_loads(rnd, buf):
        t = rnd * NW + w

        @pl.when(t < N_CHUNKS)
        def _():
            t0 = t * TQ
            pltpu.async_copy(eidx.at[1, pl.ds(t0, TQ), :], idx_v[buf],
                             sem_l[buf])
            pltpu.async_copy(att.at[pl.ds(t0, TQ), :, :],
                             slab[buf][0], sem_l[buf])
            pltpu.async_copy(att.at[pl.ds(T_TOTAL + t0, TQ), :, :],
                             slab[buf][1], sem_l[buf])

    def _wait_loads(buf):
        pltpu.make_async_copy(eidx.at[1, pl.ds(0, TQ), :],
                              idx_v[buf], sem_l[buf]).wait()
        pltpu.make_async_copy(att.at[pl.ds(0, TQ), :, :],
                              slab[buf][0], sem_l[buf]).wait()
        pltpu.make_async_copy(att.at[pl.ds(0, TQ), :, :],
                              slab[buf][1], sem_l[buf]).wait()

    def _detile(buf):
        for a in range(2):
            sl = slab[buf][a]
            tb = tbuf[buf]

            def _blk(u, carry):
                t = u // 8
                b = u - t * 8
                fcol = jnp.full((16,), 8 * a + b, jnp.int32)
                for k in range(8):
                    v = sl[t, b, pl.ds(k * 16, 16)]
                    rows = iota16 + (t * ET + k * 16)
                    plsc.store_scatter(tb, [rows, fcol], v)
                return carry

            lax.fori_loop(0, TQ * 8, _blk, 0)

    def _fire_scatters(buf):
        def _scat(q, carry):
            pltpu.async_copy(tbuf[buf].at[pl.ds(q * ET, ET), :],
                             acc.at[idx_v[buf].at[q]], sem_s, add=True)
            return carry

        lax.fori_loop(0, TQ, _scat, 0)

    def _drain_scatters(buf):
        pltpu.make_async_copy(att.at[pl.ds(0, TQ), :, :],
                              tbuf[buf], sem_s).wait()

    _fire_loads(0, 0)
    for rnd in range(MAX_ROUNDS):
        b = rnd % 2
        t = rnd * NW + w

        @pl.when(t < N_CHUNKS)
        def _():
            _wait_loads(b)
            _detile(b)

        if rnd > 0:
            tp = (rnd - 1) * NW + w

            @pl.when(tp < N_CHUNKS)
            def _():
                _drain_scatters(1 - b)

        if rnd + 1 < MAX_ROUNDS:
            _fire_loads(rnd + 1, 1 - b)

        @pl.when(t < N_CHUNKS)
        def _():
            _fire_scatters(b)

    tl = (MAX_ROUNDS - 1) * NW + w

    @pl.when(tl < N_CHUNKS)
    def _():
        _drain_scatters((MAX_ROUNDS - 1) % 2)

    plsc.subcore_barrier()
    pltpu.sync_copy(acc.at[pl.ds(s * RPS, RPS), :],
                    partials.at[c, pl.ds(s * RPS, RPS), :])


CW = 25
CR = N_NODES // CW


@functools.partial(
    pl.kernel,
    out_type=jax.ShapeDtypeStruct((N_NODES, D_EDGE), jnp.float32),
    mesh=_mesh,
    compiler_params=pltpu.CompilerParams(use_tc_tiling_on_sc=False),
    scratch_types=[
        pltpu.VMEM((CR, D_EDGE), jnp.float32),
        pltpu.VMEM((CR, D_EDGE), jnp.float32),
        pltpu.SemaphoreType.DMA,
    ],
)
def _combine(partials, out, a_v, b_v, sem):
    c = lax.axis_index("c")
    s = lax.axis_index("s")
    w = s * NC + c

    @pl.when(w < CW)
    def _():
        r0 = w * CR
        pltpu.async_copy(partials.at[0, pl.ds(r0, CR), :], a_v, sem)
        pltpu.async_copy(partials.at[1, pl.ds(r0, CR), :], b_v, sem)
        pltpu.make_async_copy(partials.at[0, pl.ds(r0, CR), :], a_v, sem).wait()
        pltpu.make_async_copy(partials.at[1, pl.ds(r0, CR), :], b_v, sem).wait()

        def _add(i, carry):
            a_v[i, :] = a_v[i, :] + b_v[i, :]
            return carry

        lax.fori_loop(0, CR, _add, 0)
        pltpu.sync_copy(a_v, out.at[pl.ds(r0, CR), :])


def kernel(x, edge_index, edge_attr, u, batch):
    eidx = edge_index.astype(jnp.int32).reshape(2, T_TOTAL, ET)
    att = (edge_attr.T.reshape(2, 8, T_TOTAL, ET)
           .transpose(0, 2, 1, 3).reshape(2 * T_TOTAL, 8, ET))
    partials = _scatter_partials(eidx, att)
    return _combine(partials)
